# baseline (speedup 1.0000x reference)
import functools
import jax
import jax.numpy as jnp
from jax import lax
from jax.experimental import pallas as pl
from jax.experimental.pallas import tpu as pltpu
from jax.experimental.pallas import tpu_sc as plsc

N = 10000
E = 320000
D = 128
H = 128
C = 16

NC = 2
NS = 16
NW = NC * NS
CH = 96
NB = 3
CPW = 108
EPW = CPW * CH
E_PAD = NW * EPW
N_ACC = 10240
ROWS_PER_TILE = N_ACC // NS


def _make_spmm(Dv):
    G = Dv // 16
    mesh = plsc.VectorSubcoreMesh(core_axis_name="c", subcore_axis_name="s")

    @functools.partial(
        pl.kernel,
        mesh=mesh,
        out_type=jax.ShapeDtypeStruct((NC, N_ACC, Dv), jnp.float32),
        scratch_types=[
            pltpu.VMEM((EPW,), jnp.int32),
            pltpu.VMEM((NB, CH), jnp.float32),
            pltpu.VMEM((CH,), jnp.int32),
            pltpu.VMEM((CH,), jnp.int32),
            pltpu.VMEM((CH,), jnp.int32),
            pltpu.VMEM((CH,), jnp.int32),
            pltpu.VMEM((CH,), jnp.int32),
            pltpu.VMEM((CH,), jnp.int32),
            pltpu.VMEM((NB, CH, Dv), jnp.float32),
            pltpu.VMEM_SHARED((N_ACC, Dv), jnp.float32),
            pltpu.SemaphoreType.DMA,
            pltpu.SemaphoreType.DMA,
            pltpu.SemaphoreType.DMA,
            pltpu.SemaphoreType.DMA,
            pltpu.SemaphoreType.DMA,
            pltpu.SemaphoreType.DMA,
            pltpu.SemaphoreType.DMA,
            pltpu.SemaphoreType.DMA,
            pltpu.SemaphoreType.DMA,
        ],
    )
    def spmm(y_hbm, sd_hbm, w_hbm, out_hbm,
             sd_st, wbuf, sc0, sc1, sc2, dc0, dc1, dc2, rows, acc,
             sem_g0, sem_g1, sem_g2, sem_s0, sem_s1, sem_s2,
             sem_w0, sem_w1, sem_w2):
        cid = lax.axis_index("c")
        sid = lax.axis_index("s")
        wid = cid * NS + sid
        ebase = wid * EPW
        r0 = sid * ROWS_PER_TILE
        scs = (sc0, sc1, sc2)
        dcs = (dc0, dc1, dc2)
        gsems = (sem_g0, sem_g1, sem_g2)
        ssems = (sem_s0, sem_s1, sem_s2)
        wsems = (sem_w0, sem_w1, sem_w2)

        pltpu.sync_copy(sd_hbm.at[pl.ds(ebase, EPW)], sd_st)

        z16 = jnp.zeros((16,), jnp.float32)

        def zrow(i, c2):
            for v in range(G):
                rows[0, i, pl.ds(v * 16, 16)] = z16
            return c2

        lax.fori_loop(0, CH, zrow, 0)
        nz = ROWS_PER_TILE // CH
        for k in range(nz):
            pltpu.sync_copy(rows.at[0],
                            acc.at[pl.ds(r0 + k * CH, CH)])
        rem = ROWS_PER_TILE - nz * CH
        if rem:
            pltpu.sync_copy(rows.at[0].at[pl.ds(0, rem)],
                            acc.at[pl.ds(r0 + nz * CH, rem)])
        plsc.subcore_barrier()

        def unpack(j, s):
            for g in range(CH // 16):
                sd = sd_st[pl.ds(j * CH + g * 16, 16)]
                sl = pl.ds(g * 16, 16)
                scs[s][sl] = sd & 0xFFFF
                dcs[s][sl] = sd >> 16

        def gather_start(j, s):
            pltpu.async_copy(y_hbm.at[scs[s]], rows.at[s], gsems[s])
            pltpu.async_copy(w_hbm.at[pl.ds(ebase + j * CH, CH)],
                             wbuf.at[s], wsems[s])

        def gather_wait(j, s):
            pltpu.make_async_copy(
                y_hbm.at[scs[s]], rows.at[s], gsems[s]).wait()
            pltpu.make_async_copy(
                w_hbm.at[pl.ds(ebase + j * CH, CH)], wbuf.at[s],
                wsems[s]).wait()

        def scatter_wait(s):
            pltpu.make_async_copy(
                rows.at[s], acc.at[dcs[s]], ssems[s]).wait()

        unpack(0, 0)
        gather_start(0, 0)

        def trip_body(jt, carry):
            for b in range(NB):
                j = jt * NB + b
                sn = (b + 1) % NB
                gather_wait(j, b)

                @pl.when(j + 1 < CPW)
                def _():
                    @pl.when(j >= NB - 1)
                    def _():
                        scatter_wait(sn)

                    unpack(j + 1, sn)
                    gather_start(j + 1, sn)

                def grp_body(g, c2):
                    wv = wbuf[b, pl.ds(g * 16, 16)]
                    for l in range(16):
                        bs = jnp.full((16,), wv[l], jnp.float32)
                        i = g * 16 + l
                        for v in range(G):
                            sl = pl.ds(v * 16, 16)
                            rows[b, i, sl] = rows[b, i, sl] * bs
                    return c2

                lax.fori_loop(0, CH // 16, grp_body, 0)
                pltpu.async_copy(rows.at[b], acc.at[dcs[b]], ssems[b],
                                 add=True)
            return carry

        lax.fori_loop(0, CPW // NB, trip_body, 0)
        for s in range(NB):
            scatter_wait(s)
        plsc.subcore_barrier()
        pltpu.sync_copy(acc.at[pl.ds(r0, ROWS_PER_TILE)],
                        out_hbm.at[cid, pl.ds(r0, ROWS_PER_TILE)])

    return spmm


_spmm_d = _make_spmm(D)


def _mm1(x, w1):
    def body(x_ref, w_ref, o_ref):
        o_ref[...] = jnp.dot(x_ref[...], w_ref[...],
                             preferred_element_type=jnp.float32)

    return pl.pallas_call(
        body,
        grid=(10,),
        in_specs=[pl.BlockSpec((N // 10, D), lambda i: (i, 0)),
                  pl.BlockSpec((D, H), lambda i: (0, 0))],
        out_specs=pl.BlockSpec((N // 10, H), lambda i: (i, 0)),
        out_shape=jax.ShapeDtypeStruct((N, H), jnp.float32),
    )(x, w1)


def _relu_mm2(p, b1, w2):
    def body(p_ref, b_ref, w_ref, o_ref):
        h = jnp.maximum(p_ref[0] + p_ref[1] + b_ref[...], 0.0)
        o_ref[...] = jnp.dot(h, w_ref[...], preferred_element_type=jnp.float32)

    return pl.pallas_call(
        body,
        grid=(10,),
        in_specs=[pl.BlockSpec((2, N // 10, H), lambda i: (0, i, 0)),
                  pl.BlockSpec((1, H), lambda i: (0, 0)),
                  pl.BlockSpec((H, 128), lambda i: (0, 0))],
        out_specs=pl.BlockSpec((N // 10, 128), lambda i: (i, 0)),
        out_shape=jax.ShapeDtypeStruct((N, 128), jnp.float32),
    )(p, b1.reshape(1, H), w2)


def _combine(p, b2):
    def body(p_ref, b_ref, o_ref):
        o_ref[...] = p_ref[0, :, :C] + p_ref[1, :, :C] + b_ref[...]

    return pl.pallas_call(
        body,
        grid=(10,),
        in_specs=[pl.BlockSpec((2, N // 10, 128), lambda i: (0, i, 0)),
                  pl.BlockSpec((1, C), lambda i: (0, 0))],
        out_specs=pl.BlockSpec((N // 10, C), lambda i: (i, 0)),
        out_shape=jax.ShapeDtypeStruct((N, C), jnp.float32),
    )(p, b2.reshape(1, C))


@jax.jit
def kernel(features, edge_index, edge_weight, W1, b1, W2, b2):
    src = edge_index[0]
    dst = edge_index[1]
    pad = E_PAD - E
    sd = jnp.bitwise_or(src, jnp.left_shift(dst, 16))
    sd_p = jnp.concatenate([sd, jnp.zeros((pad,), jnp.int32)])
    w_p = jnp.concatenate([edge_weight, jnp.zeros((pad,), jnp.float32)])
    w2p = jnp.pad(W2, ((0, 0), (0, 128 - C)))

    y1 = _mm1(features, W1)
    p1 = _spmm_d(y1, sd_p, w_p)
    y2 = _relu_mm2(p1, b1, w2p)
    p2 = _spmm_d(y2, sd_p, w_p)
    return _combine(p2, b2)


# speedup vs baseline: 1.2721x; 1.2721x over previous
import functools
import jax
import jax.numpy as jnp
from jax import lax
from jax.experimental import pallas as pl
from jax.experimental.pallas import tpu as pltpu
from jax.experimental.pallas import tpu_sc as plsc

N = 10000
E = 320000
D = 128
H = 128
C = 16

NC = 2
NS = 16
NW = NC * NS
CH = 96
NB = 2
CPW = 106
EPW = CPW * CH
E_PAD = NW * EPW
N_ACC = 10240
ROWS_PER_TILE = N_ACC // NS


def _make_spmm(Dv):
    G = Dv // 16
    mesh = plsc.VectorSubcoreMesh(core_axis_name="c", subcore_axis_name="s")

    @functools.partial(
        pl.kernel,
        mesh=mesh,
        out_type=jax.ShapeDtypeStruct((NC, N_ACC, Dv), jnp.float32),
        scratch_types=[
            pltpu.VMEM((EPW,), jnp.int32),
            pltpu.VMEM((EPW,), jnp.float32),
            pltpu.VMEM((CH,), jnp.int32),
            pltpu.VMEM((CH,), jnp.int32),
            pltpu.VMEM((CH,), jnp.int32),
            pltpu.VMEM((CH,), jnp.int32),
            pltpu.VMEM((NB, CH, Dv), jnp.float32),
            pltpu.VMEM_SHARED((N_ACC, Dv), jnp.float32),
            pltpu.SemaphoreType.DMA,
            pltpu.SemaphoreType.DMA,
            pltpu.SemaphoreType.DMA,
            pltpu.SemaphoreType.DMA,
        ],
    )
    def spmm(y_hbm, sd_hbm, w_hbm, z_hbm, out_hbm,
             sd_st, w_st, sc0, sc1, dc0, dc1, rows, acc,
             sem_g0, sem_g1, sem_s0, sem_s1):
        cid = lax.axis_index("c")
        sid = lax.axis_index("s")
        wid = cid * NS + sid
        ebase = wid * EPW
        r0 = sid * ROWS_PER_TILE
        scs = (sc0, sc1)
        dcs = (dc0, dc1)
        gsems = (sem_g0, sem_g1)
        ssems = (sem_s0, sem_s1)

        pltpu.sync_copy(sd_hbm.at[pl.ds(ebase, EPW)], sd_st)
        pltpu.sync_copy(w_hbm.at[pl.ds(ebase, EPW)], w_st)
        pltpu.sync_copy(z_hbm, acc.at[pl.ds(r0, ROWS_PER_TILE)])
        plsc.subcore_barrier()

        def unpack(j, s):
            for g in range(CH // 16):
                sd = sd_st[pl.ds(j * CH + g * 16, 16)]
                sl = pl.ds(g * 16, 16)
                scs[s][sl] = sd & 0xFFFF
                dcs[s][sl] = sd >> 16

        def gather_start(j, s):
            pltpu.async_copy(y_hbm.at[scs[s]], rows.at[s], gsems[s])

        def gather_wait(j, s):
            pltpu.make_async_copy(
                y_hbm.at[scs[s]], rows.at[s], gsems[s]).wait()

        def scatter_wait(s):
            pltpu.make_async_copy(
                rows.at[s], acc.at[dcs[s]], ssems[s]).wait()

        unpack(0, 0)
        gather_start(0, 0)

        def trip_body(jt, carry):
            for b in range(NB):
                j = jt * NB + b
                sn = (b + 1) % NB
                gather_wait(j, b)

                @pl.when(j + 1 < CPW)
                def _():
                    @pl.when(j >= NB - 1)
                    def _():
                        scatter_wait(sn)

                    unpack(j + 1, sn)
                    gather_start(j + 1, sn)

                def grp_body(g, c2):
                    wv = w_st[pl.ds(j * CH + g * 16, 16)]
                    for l in range(16):
                        bs = jnp.full((16,), wv[l], jnp.float32)
                        i = g * 16 + l
                        for v in range(G):
                            sl = pl.ds(v * 16, 16)
                            rows[b, i, sl] = rows[b, i, sl] * bs
                    return c2

                lax.fori_loop(0, CH // 16, grp_body, 0)
                pltpu.async_copy(rows.at[b], acc.at[dcs[b]], ssems[b],
                                 add=True)
            return carry

        lax.fori_loop(0, CPW // NB, trip_body, 0)
        for s in range(NB):
            scatter_wait(s)
        plsc.subcore_barrier()
        pltpu.sync_copy(acc.at[pl.ds(r0, ROWS_PER_TILE)],
                        out_hbm.at[cid, pl.ds(r0, ROWS_PER_TILE)])

    return spmm


_spmm_d = _make_spmm(D)


def _mm1(x, w1):
    def body(x_ref, w_ref, o_ref):
        o_ref[...] = jnp.dot(x_ref[...], w_ref[...],
                             preferred_element_type=jnp.float32)

    return pl.pallas_call(
        body,
        grid=(10,),
        in_specs=[pl.BlockSpec((N // 10, D), lambda i: (i, 0)),
                  pl.BlockSpec((D, H), lambda i: (0, 0))],
        out_specs=pl.BlockSpec((N // 10, H), lambda i: (i, 0)),
        out_shape=jax.ShapeDtypeStruct((N, H), jnp.float32),
    )(x, w1)


def _relu_mm2(p, b1, w2):
    def body(p_ref, b_ref, w_ref, o_ref):
        h = jnp.maximum(p_ref[0] + p_ref[1] + b_ref[...], 0.0)
        o_ref[...] = jnp.dot(h, w_ref[...], preferred_element_type=jnp.float32)

    return pl.pallas_call(
        body,
        grid=(10,),
        in_specs=[pl.BlockSpec((2, N // 10, H), lambda i: (0, i, 0)),
                  pl.BlockSpec((1, H), lambda i: (0, 0)),
                  pl.BlockSpec((H, 128), lambda i: (0, 0))],
        out_specs=pl.BlockSpec((N // 10, 128), lambda i: (i, 0)),
        out_shape=jax.ShapeDtypeStruct((N, 128), jnp.float32),
    )(p, b1.reshape(1, H), w2)


def _combine(p, b2):
    def body(p_ref, b_ref, o_ref):
        o_ref[...] = p_ref[0, :, :C] + p_ref[1, :, :C] + b_ref[...]

    return pl.pallas_call(
        body,
        grid=(10,),
        in_specs=[pl.BlockSpec((2, N // 10, 128), lambda i: (0, i, 0)),
                  pl.BlockSpec((1, C), lambda i: (0, 0))],
        out_specs=pl.BlockSpec((N // 10, C), lambda i: (i, 0)),
        out_shape=jax.ShapeDtypeStruct((N, C), jnp.float32),
    )(p, b2.reshape(1, C))


@jax.jit
def kernel(features, edge_index, edge_weight, W1, b1, W2, b2):
    src = edge_index[0]
    dst = edge_index[1]
    pad = E_PAD - E
    sd = jnp.bitwise_or(src, jnp.left_shift(dst, 16))
    sd_p = jnp.concatenate([sd, jnp.zeros((pad,), jnp.int32)])
    w_p = jnp.concatenate([edge_weight, jnp.zeros((pad,), jnp.float32)])
    w2p = jnp.pad(W2, ((0, 0), (0, 128 - C)))
    z_d = jnp.zeros((ROWS_PER_TILE, D), jnp.float32)

    y1 = _mm1(features, W1)
    p1 = _spmm_d(y1, sd_p, w_p, z_d)
    y2 = _relu_mm2(p1, b1, w2p)
    p2 = _spmm_d(y2, sd_p, w_p, z_d)
    return _combine(p2, b2)


# speedup vs baseline: 1.5549x; 1.2223x over previous
import functools
import jax
import jax.numpy as jnp
from jax import lax
from jax.experimental import pallas as pl
from jax.experimental.pallas import tpu as pltpu
from jax.experimental.pallas import tpu_sc as plsc

N = 10000
E = 320000
D = 128
H = 128
C = 16

NC = 2
NS = 16
NW = NC * NS
CH = 96
NB = 2
CPW = 106
EPW = CPW * CH
N_ACC = 10240
ROWS_PER_TILE = N_ACC // NS

CH2 = 128
NB2 = 3
CPW2 = 81
EPW2 = CPW2 * CH2
E_PAD = NW * EPW2


def _make_spmm(Dv, g_active=None):
    G = g_active if g_active is not None else Dv // 16
    mesh = plsc.VectorSubcoreMesh(core_axis_name="c", subcore_axis_name="s")

    @functools.partial(
        pl.kernel,
        mesh=mesh,
        out_type=jax.ShapeDtypeStruct((NC, N_ACC, Dv), jnp.float32),
        scratch_types=[
            pltpu.VMEM((EPW,), jnp.int32),
            pltpu.VMEM((EPW,), jnp.float32),
            pltpu.VMEM((CH,), jnp.int32),
            pltpu.VMEM((CH,), jnp.int32),
            pltpu.VMEM((CH,), jnp.int32),
            pltpu.VMEM((CH,), jnp.int32),
            pltpu.VMEM((NB, CH, Dv), jnp.float32),
            pltpu.VMEM_SHARED((N_ACC, Dv), jnp.float32),
            pltpu.SemaphoreType.DMA,
            pltpu.SemaphoreType.DMA,
            pltpu.SemaphoreType.DMA,
            pltpu.SemaphoreType.DMA,
        ],
    )
    def spmm(y_hbm, sd_hbm, w_hbm, z_hbm, out_hbm,
             sd_st, w_st, sc0, sc1, dc0, dc1, rows, acc,
             sem_g0, sem_g1, sem_s0, sem_s1):
        cid = lax.axis_index("c")
        sid = lax.axis_index("s")
        wid = cid * NS + sid
        ebase = wid * EPW
        r0 = sid * ROWS_PER_TILE
        scs = (sc0, sc1)
        dcs = (dc0, dc1)
        gsems = (sem_g0, sem_g1)
        ssems = (sem_s0, sem_s1)

        pltpu.sync_copy(sd_hbm.at[pl.ds(ebase, EPW)], sd_st)
        pltpu.sync_copy(w_hbm.at[pl.ds(ebase, EPW)], w_st)
        pltpu.sync_copy(z_hbm, acc.at[pl.ds(r0, ROWS_PER_TILE)])
        plsc.subcore_barrier()

        def unpack(j, s):
            for g in range(CH // 16):
                sd = sd_st[pl.ds(j * CH + g * 16, 16)]
                sl = pl.ds(g * 16, 16)
                scs[s][sl] = sd & 0xFFFF
                dcs[s][sl] = sd >> 16

        def gather_start(j, s):
            pltpu.async_copy(y_hbm.at[scs[s]], rows.at[s], gsems[s])

        def gather_wait(j, s):
            pltpu.make_async_copy(
                y_hbm.at[scs[s]], rows.at[s], gsems[s]).wait()

        def scatter_wait(s):
            pltpu.make_async_copy(
                rows.at[s], acc.at[dcs[s]], ssems[s]).wait()

        unpack(0, 0)
        gather_start(0, 0)

        def trip_body(jt, carry):
            for b in range(NB):
                j = jt * NB + b
                sn = (b + 1) % NB
                gather_wait(j, b)

                @pl.when(j + 1 < CPW)
                def _():
                    @pl.when(j >= NB - 1)
                    def _():
                        scatter_wait(sn)

                    unpack(j + 1, sn)
                    gather_start(j + 1, sn)

                def grp_body(g, c2):
                    wv = w_st[pl.ds(j * CH + g * 16, 16)]
                    for l in range(16):
                        bs = jnp.full((16,), wv[l], jnp.float32)
                        i = g * 16 + l
                        for v in range(G):
                            sl = pl.ds(v * 16, 16)
                            rows[b, i, sl] = rows[b, i, sl] * bs
                    return c2

                lax.fori_loop(0, CH // 16, grp_body, 0)
                pltpu.async_copy(rows.at[b], acc.at[dcs[b]], ssems[b],
                                 add=True)
            return carry

        lax.fori_loop(0, CPW // NB, trip_body, 0)
        for s in range(NB):
            scatter_wait(s)
        plsc.subcore_barrier()
        pltpu.sync_copy(acc.at[pl.ds(r0, ROWS_PER_TILE)],
                        out_hbm.at[cid, pl.ds(r0, ROWS_PER_TILE)])

    return spmm


_spmm_d = _make_spmm(D)


_spmm_d2 = _make_spmm(D, g_active=1)


def _mm1(x, w1):
    def body(x_ref, w_ref, o_ref):
        o_ref[...] = jnp.dot(x_ref[...], w_ref[...],
                             preferred_element_type=jnp.float32)

    return pl.pallas_call(
        body,
        grid=(10,),
        in_specs=[pl.BlockSpec((N // 10, D), lambda i: (i, 0)),
                  pl.BlockSpec((D, H), lambda i: (0, 0))],
        out_specs=pl.BlockSpec((N // 10, H), lambda i: (i, 0)),
        out_shape=jax.ShapeDtypeStruct((N, H), jnp.float32),
    )(x, w1)


def _relu_mm2(p, b1, w2):
    def body(p_ref, b_ref, w_ref, o_ref):
        h = jnp.maximum(p_ref[0] + p_ref[1] + b_ref[...], 0.0)
        o_ref[...] = jnp.dot(h, w_ref[...], preferred_element_type=jnp.float32)

    return pl.pallas_call(
        body,
        grid=(10,),
        in_specs=[pl.BlockSpec((2, N // 10, H), lambda i: (0, i, 0)),
                  pl.BlockSpec((1, H), lambda i: (0, 0)),
                  pl.BlockSpec((H, 128), lambda i: (0, 0))],
        out_specs=pl.BlockSpec((N // 10, 128), lambda i: (i, 0)),
        out_shape=jax.ShapeDtypeStruct((N, 128), jnp.float32),
    )(p, b1.reshape(1, H), w2)


def _combine(p, b2):
    def body(p_ref, b_ref, o_ref):
        o_ref[...] = p_ref[0, :, :C] + p_ref[1, :, :C] + b_ref[...]

    return pl.pallas_call(
        body,
        grid=(10,),
        in_specs=[pl.BlockSpec((2, N // 10, 128), lambda i: (0, i, 0)),
                  pl.BlockSpec((1, C), lambda i: (0, 0))],
        out_specs=pl.BlockSpec((N // 10, C), lambda i: (i, 0)),
        out_shape=jax.ShapeDtypeStruct((N, C), jnp.float32),
    )(p, b2.reshape(1, C))


@jax.jit
def kernel(features, edge_index, edge_weight, W1, b1, W2, b2):
    src = edge_index[0]
    dst = edge_index[1]
    pad = E_PAD - E
    sd = jnp.bitwise_or(src, jnp.left_shift(dst, 16))
    sd_p = jnp.concatenate([sd, jnp.zeros((pad,), jnp.int32)])
    w_p = jnp.concatenate([edge_weight, jnp.zeros((pad,), jnp.float32)])
    z_d = jnp.zeros((ROWS_PER_TILE, D), jnp.float32)
    w2p = jnp.pad(W2, ((0, 0), (0, 128 - C)))

    y1 = _mm1(features, W1)
    p1 = _spmm_d(y1, sd_p, w_p, z_d)
    y2 = _relu_mm2(p1, b1, w2p)
    p2 = _spmm_d2(y2, sd_p, w_p, z_d)
    return _combine(p2, b2)


# speedup vs baseline: 1.5561x; 1.0008x over previous
import functools
import jax
import jax.numpy as jnp
from jax import lax
from jax.experimental import pallas as pl
from jax.experimental.pallas import tpu as pltpu
from jax.experimental.pallas import tpu_sc as plsc

N = 10000
E = 320000
D = 128
H = 128
C = 16

NC = 2
NS = 16
NW = NC * NS
CH = 96
NB = 2
CPW = 106
EPW = CPW * CH
N_ACC = 10240
ROWS_PER_TILE = N_ACC // NS

CH2 = 128
NB2 = 3
CPW2 = 81
EPW2 = CPW2 * CH2
E_PAD = NW * EPW2


def _make_spmm(Dv, g_active=None):
    G = g_active if g_active is not None else Dv // 16
    mesh = plsc.VectorSubcoreMesh(core_axis_name="c", subcore_axis_name="s")

    @functools.partial(
        pl.kernel,
        mesh=mesh,
        out_type=jax.ShapeDtypeStruct((NC, N_ACC, Dv), jnp.float32),
        scratch_types=[
            pltpu.VMEM((EPW,), jnp.int32),
            pltpu.VMEM((EPW,), jnp.float32),
            pltpu.VMEM((CH,), jnp.int32),
            pltpu.VMEM((CH,), jnp.int32),
            pltpu.VMEM((CH,), jnp.int32),
            pltpu.VMEM((CH,), jnp.int32),
            pltpu.VMEM((NB, CH, Dv), jnp.float32),
            pltpu.VMEM_SHARED((N_ACC, Dv), jnp.float32),
            pltpu.SemaphoreType.DMA,
            pltpu.SemaphoreType.DMA,
            pltpu.SemaphoreType.DMA,
            pltpu.SemaphoreType.DMA,
        ],
    )
    def spmm(y_hbm, sd_hbm, w_hbm, z_hbm, out_hbm,
             sd_st, w_st, sc0, sc1, dc0, dc1, rows, acc,
             sem_g0, sem_g1, sem_s0, sem_s1):
        cid = lax.axis_index("c")
        sid = lax.axis_index("s")
        wid = cid * NS + sid
        ebase = wid * EPW
        r0 = sid * ROWS_PER_TILE
        scs = (sc0, sc1)
        dcs = (dc0, dc1)
        gsems = (sem_g0, sem_g1)
        ssems = (sem_s0, sem_s1)

        pltpu.sync_copy(sd_hbm.at[pl.ds(ebase, EPW)], sd_st)
        pltpu.sync_copy(w_hbm.at[pl.ds(ebase, EPW)], w_st)
        pltpu.sync_copy(z_hbm, acc.at[pl.ds(r0, ROWS_PER_TILE)])
        plsc.subcore_barrier()

        def unpack(j, s):
            for g in range(CH // 16):
                sd = sd_st[pl.ds(j * CH + g * 16, 16)]
                sl = pl.ds(g * 16, 16)
                scs[s][sl] = sd & 0xFFFF
                dcs[s][sl] = sd >> 16

        def gather_start(j, s):
            pltpu.async_copy(y_hbm.at[scs[s]], rows.at[s], gsems[s])

        def gather_wait(j, s):
            pltpu.make_async_copy(
                y_hbm.at[scs[s]], rows.at[s], gsems[s]).wait()

        def scatter_wait(s):
            pltpu.make_async_copy(
                rows.at[s], acc.at[dcs[s]], ssems[s]).wait()

        unpack(0, 0)
        gather_start(0, 0)

        def trip_body(jt, carry):
            for b in range(NB):
                j = jt * NB + b
                sn = (b + 1) % NB
                gather_wait(j, b)

                @pl.when(j + 1 < CPW)
                def _():
                    @pl.when(j >= NB - 1)
                    def _():
                        scatter_wait(sn)

                    unpack(j + 1, sn)
                    gather_start(j + 1, sn)

                def grp_body(g, c2):
                    wv = w_st[pl.ds(j * CH + g * 16, 16)]
                    bss = [jnp.full((16,), wv[l], jnp.float32)
                           for l in range(16)]
                    for l0 in range(0, 16, 4):
                        vals = [[rows[b, g * 16 + l0 + q, pl.ds(v * 16, 16)]
                                 for v in range(G)] for q in range(4)]
                        for q in range(4):
                            for v in range(G):
                                rows[b, g * 16 + l0 + q, pl.ds(v * 16, 16)] = (
                                    vals[q][v] * bss[l0 + q])
                    return c2

                lax.fori_loop(0, CH // 16, grp_body, 0)
                pltpu.async_copy(rows.at[b], acc.at[dcs[b]], ssems[b],
                                 add=True)
            return carry

        lax.fori_loop(0, CPW // NB, trip_body, 0)
        for s in range(NB):
            scatter_wait(s)
        plsc.subcore_barrier()
        pltpu.sync_copy(acc.at[pl.ds(r0, ROWS_PER_TILE)],
                        out_hbm.at[cid, pl.ds(r0, ROWS_PER_TILE)])

    return spmm


_spmm_d = _make_spmm(D)


_spmm_d2 = _make_spmm(D, g_active=1)


def _mm1(x, w1):
    def body(x_ref, w_ref, o_ref):
        o_ref[...] = jnp.dot(x_ref[...], w_ref[...],
                             preferred_element_type=jnp.float32)

    return pl.pallas_call(
        body,
        grid=(10,),
        in_specs=[pl.BlockSpec((N // 10, D), lambda i: (i, 0)),
                  pl.BlockSpec((D, H), lambda i: (0, 0))],
        out_specs=pl.BlockSpec((N // 10, H), lambda i: (i, 0)),
        out_shape=jax.ShapeDtypeStruct((N, H), jnp.float32),
    )(x, w1)


def _relu_mm2(p, b1, w2):
    def body(p_ref, b_ref, w_ref, o_ref):
        h = jnp.maximum(p_ref[0] + p_ref[1] + b_ref[...], 0.0)
        o_ref[...] = jnp.dot(h, w_ref[...], preferred_element_type=jnp.float32)

    return pl.pallas_call(
        body,
        grid=(10,),
        in_specs=[pl.BlockSpec((2, N // 10, H), lambda i: (0, i, 0)),
                  pl.BlockSpec((1, H), lambda i: (0, 0)),
                  pl.BlockSpec((H, 128), lambda i: (0, 0))],
        out_specs=pl.BlockSpec((N // 10, 128), lambda i: (i, 0)),
        out_shape=jax.ShapeDtypeStruct((N, 128), jnp.float32),
    )(p, b1.reshape(1, H), w2)


def _combine(p, b2):
    def body(p_ref, b_ref, o_ref):
        o_ref[...] = p_ref[0, :, :C] + p_ref[1, :, :C] + b_ref[...]

    return pl.pallas_call(
        body,
        grid=(10,),
        in_specs=[pl.BlockSpec((2, N // 10, 128), lambda i: (0, i, 0)),
                  pl.BlockSpec((1, C), lambda i: (0, 0))],
        out_specs=pl.BlockSpec((N // 10, C), lambda i: (i, 0)),
        out_shape=jax.ShapeDtypeStruct((N, C), jnp.float32),
    )(p, b2.reshape(1, C))


@jax.jit
def kernel(features, edge_index, edge_weight, W1, b1, W2, b2):
    src = edge_index[0]
    dst = edge_index[1]
    pad = E_PAD - E
    sd = jnp.bitwise_or(src, jnp.left_shift(dst, 16))
    sd_p = jnp.concatenate([sd, jnp.zeros((pad,), jnp.int32)])
    w_p = jnp.concatenate([edge_weight, jnp.zeros((pad,), jnp.float32)])
    z_d = jnp.zeros((ROWS_PER_TILE, D), jnp.float32)
    w2p = jnp.pad(W2, ((0, 0), (0, 128 - C)))

    y1 = _mm1(features, W1)
    p1 = _spmm_d(y1, sd_p, w_p, z_d)
    y2 = _relu_mm2(p1, b1, w2p)
    p2 = _spmm_d2(y2, sd_p, w_p, z_d)
    return _combine(p2, b2)
